# baseline (speedup 1.0000x reference)
import functools

import jax
import jax.numpy as jnp
from jax import lax
from jax.experimental import pallas as pl
from jax.experimental.pallas import tpu as pltpu
from jax.experimental.pallas import tpu_sc as plsc

M, N = 128, 32768
NTOT = M * N
K = (NTOT * 10) // 100

NB = 16384
NBX = NB + 16
SH1 = 18
SH2 = 4
NW = 32
CH = N

_MESH = plsc.VectorSubcoreMesh(core_axis_name="c", subcore_axis_name="s")
_SC_PARAMS = pltpu.CompilerParams(needs_layout_passes=False)


def _loss_body(x_ref, t_ref, o_ref):
    x = x_ref[...]
    t = t_ref[...]
    o_ref[...] = (1.0 - t) * x + (
        jnp.log1p(jnp.exp(-jnp.abs(x))) + jnp.maximum(-x, 0.0)
    )


def _compute_loss(pred, tgt, rows, half):
    blk = 4096
    return pl.pallas_call(
        _loss_body,
        grid=(N // blk,),
        in_specs=[
            pl.BlockSpec((rows, blk), lambda i, h=half: (h, i)),
            pl.BlockSpec((rows, blk), lambda i, h=half: (h, i)),
        ],
        out_specs=pl.BlockSpec((rows, blk), lambda i: (0, i)),
        out_shape=jax.ShapeDtypeStruct((rows, N), jnp.float32),
    )(pred, tgt)


def _stream(chunks, buf0, buf1, sem0, sem1, process):
    for i, ch in enumerate(chunks):
        b, s = (buf0, sem0) if i % 2 == 0 else (buf1, sem1)
        if i + 1 < len(chunks):
            nb, ns = (buf1, sem1) if i % 2 == 0 else (buf0, sem0)
            pltpu.async_copy(chunks[i + 1], nb, ns)
        pltpu.make_async_copy(chunks[0], b, s).wait()
        process(b)


def _make_sc_hist(rows):
    rpt = rows // NW

    @functools.partial(
        pl.kernel,
        out_type=jax.ShapeDtypeStruct((NW, NB), jnp.int32),
        mesh=_MESH,
        compiler_params=_SC_PARAMS,
        scratch_types=[
            pltpu.VMEM((CH,), jnp.float32),
            pltpu.VMEM((CH,), jnp.float32),
            pltpu.VMEM((NB,), jnp.int32),
            pltpu.SemaphoreType.DMA,
            pltpu.SemaphoreType.DMA,
        ],
    )
    def sc_hist(loss_hbm, out_hbm, buf0, buf1, hist, sem0, sem1):
        wid = lax.axis_index("s") * 2 + lax.axis_index("c")
        row0 = wid * rpt
        chunks = [loss_hbm.at[row0 + r] for r in range(rpt)]
        pltpu.async_copy(chunks[0], buf0, sem0)
        zi = jnp.zeros((16,), jnp.int32)

        @plsc.parallel_loop(0, NB // 16, unroll=8)
        def _(i):
            hist[pl.ds(i * 16, 16)] = zi

        ones = jnp.ones((16,), jnp.int32)
        sh1 = jnp.full((16,), SH1, jnp.int32)

        def process(buf):
            @plsc.parallel_loop(0, CH // 16, unroll=8)
            def _(j):
                v = buf[pl.ds(j * 16, 16)]
                u = plsc.bitcast(v, jnp.int32)
                key = jnp.right_shift(u, sh1)
                plsc.addupdate_scatter(hist, [key], ones)

        _stream(chunks, buf0, buf1, sem0, sem1, process)
        pltpu.sync_copy(hist, out_hbm.at[wid])

    return sc_hist


def _make_select1(nh):
    def body(*refs):
        hist_refs, (bvec_ref, meta_ref) = refs[:nh], refs[nh:]
        cs = hist_refs[0][...].sum(axis=0)
        for h in hist_refs[1:]:
            cs = cs + h[...].sum(axis=0)
        idx = lax.broadcasted_iota(jnp.int32, (NB,), 0)

        def srch(_, lohi):
            lo, hi = lohi
            mid = (lo + hi) // 2
            s = jnp.sum(jnp.where(idx >= mid, cs, 0))
            big = s >= K
            return jnp.where(big, mid, lo), jnp.where(big, hi, mid)

        bstar, _ = lax.fori_loop(0, 14, srch, (jnp.int32(0), jnp.int32(NB)))
        cgt = jnp.sum(jnp.where(idx > bstar, cs, 0))
        bvec_ref[...] = jnp.broadcast_to(bstar, (1, 16)).astype(jnp.int32)
        lanes = lax.broadcasted_iota(jnp.int32, (1, 16), 1)
        meta_ref[...] = jnp.where(
            lanes == 0, bstar, jnp.where(lanes == 1, cgt, 0))

    def run(*hists):
        return pl.pallas_call(
            body,
            out_shape=(
                jax.ShapeDtypeStruct((1, 16), jnp.int32),
                jax.ShapeDtypeStruct((1, 16), jnp.int32),
            ),
        )(*hists)

    return run


def _make_sc_refine(nh, rows):
    rpt = rows // NW

    def body(*refs):
        loss_refs = refs[:nh]
        bvec_hbm, cnt_out, sum_out = refs[nh:nh + 3]
        buf0, buf1, cnt2, sum2, bv_v, sem0, sem1 = refs[nh + 3:]
        wid = lax.axis_index("s") * 2 + lax.axis_index("c")
        row0 = wid * rpt
        chunks = [h.at[row0 + r] for h in loss_refs for r in range(rpt)]
        pltpu.async_copy(chunks[0], buf0, sem0)
        zi = jnp.zeros((16,), jnp.int32)
        zf = jnp.zeros((16,), jnp.float32)

        @plsc.parallel_loop(0, NBX // 16, unroll=8)
        def _(i):
            sum2[pl.ds(i * 16, 16)] = zf

        @plsc.parallel_loop(0, NB // 16, unroll=8)
        def _(i):
            cnt2[pl.ds(i * 16, 16)] = zi

        pltpu.sync_copy(bvec_hbm.at[0], bv_v)
        bv = bv_v[...]
        ones = jnp.ones((16,), jnp.int32)
        sh1 = jnp.full((16,), SH1, jnp.int32)
        sh2 = jnp.full((16,), SH2, jnp.int32)
        msk = jnp.full((16,), NB - 1, jnp.int32)
        oflow = jnp.full((16,), NB, jnp.int32) + lax.iota(jnp.int32, 16)

        def process(buf):
            @plsc.parallel_loop(0, CH // 16, unroll=8)
            def _(j):
                v = buf[pl.ds(j * 16, 16)]
                u = plsc.bitcast(v, jnp.int32)
                k1 = jnp.right_shift(u, sh1)
                m_eq = k1 == bv
                k2 = jnp.bitwise_and(jnp.right_shift(u, sh2), msk)
                plsc.addupdate_scatter(cnt2, [k2], ones, mask=m_eq)
                ks = jnp.where(m_eq, k2, oflow)
                plsc.addupdate_scatter(sum2, [ks], v, mask=k1 >= bv)

        _stream(chunks, buf0, buf1, sem0, sem1, process)
        pltpu.sync_copy(cnt2, cnt_out.at[wid])
        pltpu.sync_copy(sum2, sum_out.at[wid])

    return functools.partial(
        pl.kernel,
        out_type=(
            jax.ShapeDtypeStruct((NW, NB), jnp.int32),
            jax.ShapeDtypeStruct((NW, NBX), jnp.float32),
        ),
        mesh=_MESH,
        compiler_params=_SC_PARAMS,
        scratch_types=[
            pltpu.VMEM((CH,), jnp.float32),
            pltpu.VMEM((CH,), jnp.float32),
            pltpu.VMEM((NB,), jnp.int32),
            pltpu.VMEM((NBX,), jnp.float32),
            pltpu.VMEM((16,), jnp.int32),
            pltpu.SemaphoreType.DMA,
            pltpu.SemaphoreType.DMA,
        ],
    )(body)


NB1 = 32768
SH0 = 17
CH1 = 16384


@functools.partial(
    pl.kernel,
    out_type=(
        jax.ShapeDtypeStruct((NW, NB1), jnp.int32),
        jax.ShapeDtypeStruct((NW, NB1), jnp.float32),
    ),
    mesh=_MESH,
    compiler_params=_SC_PARAMS,
    scratch_types=[
        pltpu.VMEM((CH1,), jnp.float32),
        pltpu.VMEM((CH1,), jnp.float32),
        pltpu.VMEM((NB1,), jnp.int32),
        pltpu.VMEM((NB1,), jnp.float32),
        pltpu.SemaphoreType.DMA,
        pltpu.SemaphoreType.DMA,
    ],
)
def _sc_histsum(loss_hbm, cnt_out, sum_out, buf0, buf1, cnt, sm, sem0, sem1):
    wid = lax.axis_index("s") * 2 + lax.axis_index("c")
    row0 = wid * (M // NW)
    chunks = [loss_hbm.at[row0 + r, pl.ds(h * CH1, CH1)]
              for r in range(M // NW) for h in range(N // CH1)]
    pltpu.async_copy(chunks[0], buf0, sem0)
    zi = jnp.zeros((16,), jnp.int32)
    zf = jnp.zeros((16,), jnp.float32)

    @plsc.parallel_loop(0, NB1 // 16, unroll=8)
    def _(i):
        cnt[pl.ds(i * 16, 16)] = zi
        sm[pl.ds(i * 16, 16)] = zf

    ones = jnp.ones((16,), jnp.int32)
    sh0 = jnp.full((16,), SH0, jnp.int32)

    def process(buf):
        @plsc.parallel_loop(0, CH1 // 16, unroll=16)
        def _(j):
            v = buf[pl.ds(j * 16, 16)]
            u = plsc.bitcast(v, jnp.int32)
            key = jnp.right_shift(u, sh0)
            plsc.addupdate_scatter(cnt, [key], ones)
            plsc.addupdate_scatter(sm, [key], v)

    _stream(chunks, buf0, buf1, sem0, sem1, process)
    pltpu.async_copy(cnt, cnt_out.at[wid], sem0)
    pltpu.async_copy(sm, sum_out.at[wid], sem1)
    pltpu.make_async_copy(cnt, cnt_out.at[wid], sem0).wait()
    pltpu.make_async_copy(sm, sum_out.at[wid], sem1).wait()


CH2 = 8192
_LOG1P_C = (
    3.3869653288105303e-08, 0.9999942724811816, -0.4998385618342832,
    0.331548616520496, -0.23982616050283964, 0.16582275268894042,
    -0.09325203898483149, 0.03484971247815497, -0.006151470961746543,
)


@functools.partial(
    pl.kernel,
    out_type=(
        jax.ShapeDtypeStruct((NW, NB1), jnp.int32),
        jax.ShapeDtypeStruct((NW, NB1), jnp.float32),
    ),
    mesh=_MESH,
    compiler_params=_SC_PARAMS,
    scratch_types=[
        pltpu.VMEM((CH2,), jnp.float32),
        pltpu.VMEM((CH2,), jnp.float32),
        pltpu.VMEM((CH2,), jnp.float32),
        pltpu.VMEM((CH2,), jnp.float32),
        pltpu.VMEM((NB1,), jnp.int32),
        pltpu.VMEM((NB1,), jnp.float32),
        pltpu.SemaphoreType.DMA,
        pltpu.SemaphoreType.DMA,
    ],
)
def _sc_losshist(pred_hbm, tgt_hbm, cnt_out, sum_out,
                 bx0, bt0, bx1, bt1, cnt, sm, sem0, sem1):
    wid = lax.axis_index("s") * 2 + lax.axis_index("c")
    row0 = wid * (M // NW)
    pairs = [(pred_hbm.at[row0 + r, pl.ds(h * CH2, CH2)],
              tgt_hbm.at[row0 + r, pl.ds(h * CH2, CH2)])
             for r in range(M // NW) for h in range(N // CH2)]

    def issue(pair, bx, bt, sem):
        pltpu.async_copy(pair[0], bx, sem)
        pltpu.async_copy(pair[1], bt, sem)

    def drain(pair, bx, bt, sem):
        pltpu.make_async_copy(pair[0], bx, sem).wait()
        pltpu.make_async_copy(pair[1], bt, sem).wait()

    issue(pairs[0], bx0, bt0, sem0)
    zi = jnp.zeros((16,), jnp.int32)
    zf = jnp.zeros((16,), jnp.float32)

    @plsc.parallel_loop(0, NB1 // 16, unroll=8)
    def _(i):
        cnt[pl.ds(i * 16, 16)] = zi
        sm[pl.ds(i * 16, 16)] = zf

    ones = jnp.ones((16,), jnp.int32)
    sh0 = jnp.full((16,), SH0, jnp.int32)

    def process(bx, bt):
        @plsc.parallel_loop(0, CH2 // 16, unroll=8)
        def _(j):
            x = bx[pl.ds(j * 16, 16)]
            t = bt[pl.ds(j * 16, 16)]
            y = jnp.exp(-jnp.abs(x))
            p = jnp.full((16,), _LOG1P_C[-1], jnp.float32)
            for c in reversed(_LOG1P_C[:-1]):
                p = p * y + c
            v = (1.0 - t) * x + jnp.maximum(-x, 0.0) + p
            v = jnp.maximum(v, 0.0)
            u = plsc.bitcast(v, jnp.int32)
            key = jnp.right_shift(u, sh0)
            plsc.addupdate_scatter(cnt, [key], ones)
            plsc.addupdate_scatter(sm, [key], v)

    bufs = [(bx0, bt0, sem0), (bx1, bt1, sem1)]
    for i, pair in enumerate(pairs):
        bx, bt, sem = bufs[i % 2]
        if i + 1 < len(pairs):
            nbx, nbt, nsem = bufs[(i + 1) % 2]
            issue(pairs[i + 1], nbx, nbt, nsem)
        drain(pair, bx, bt, sem)
        process(bx, bt)

    pltpu.async_copy(cnt, cnt_out.at[wid], sem0)
    pltpu.async_copy(sm, sum_out.at[wid], sem1)
    pltpu.make_async_copy(cnt, cnt_out.at[wid], sem0).wait()
    pltpu.make_async_copy(sm, sum_out.at[wid], sem1).wait()


def _finalize1_body(cnt_ref, sum_ref, out_ref):
    cnt = jnp.sum(cnt_ref[...], axis=0)
    sm = jnp.sum(sum_ref[...], axis=0)
    idx = lax.broadcasted_iota(jnp.int32, (NB1,), 0)

    def srch(_, lohi):
        lo, hi = lohi
        mid = (lo + hi) // 2
        s = jnp.sum(jnp.where(idx >= mid, cnt, 0))
        big = s >= K
        return jnp.where(big, mid, lo), jnp.where(big, hi, mid)

    bstar, _ = lax.fori_loop(0, 15, srch, (jnp.int32(0), jnp.int32(NB1)))
    cgt = jnp.sum(jnp.where(idx > bstar, cnt, 0))
    sum_gt = jnp.sum(jnp.where(idx > bstar, sm, 0.0))
    nb = jnp.sum(jnp.where(idx == bstar, cnt, 0)).astype(jnp.float32)
    r = (K - cgt).astype(jnp.float32)
    lo_arr = lax.bitcast_convert_type(
        jnp.full((1, 1), 0, jnp.int32) + jnp.left_shift(bstar, SH0),
        jnp.float32)
    hi_arr = lax.bitcast_convert_type(
        jnp.full((1, 1), 0, jnp.int32) + jnp.left_shift(bstar + 1, SH0),
        jnp.float32)
    est = hi_arr - (r / (2.0 * nb)) * (hi_arr - lo_arr)
    out_ref[...] = (sum_gt + r * est) * jnp.float32(1.0 / K)


def _finalize1(cnt, sm):
    return pl.pallas_call(
        _finalize1_body,
        out_shape=jax.ShapeDtypeStruct((1, 1), jnp.float32),
    )(cnt, sm)


def _finalize_body(cnt_ref, sum_ref, meta_ref, out_ref):
    cnt = jnp.sum(cnt_ref[...], axis=0)
    sm = jnp.sum(sum_ref[...], axis=0)
    meta = meta_ref[...]
    bstar = meta[0, 0]
    r = K - meta[0, 1]
    idx = lax.broadcasted_iota(jnp.int32, (NB,), 0)
    idxx = lax.broadcasted_iota(jnp.int32, (NBX,), 0)

    def srch(_, lohi):
        lo, hi = lohi
        mid = (lo + hi) // 2
        s = jnp.sum(jnp.where(idx >= mid, cnt, 0))
        big = s >= r
        return jnp.where(big, mid, lo), jnp.where(big, hi, mid)

    sstar, _ = lax.fori_loop(0, 14, srch, (jnp.int32(0), jnp.int32(NB)))
    cgt2 = jnp.sum(jnp.where(idx > sstar, cnt, 0))
    sgt2 = jnp.sum(jnp.where((idxx > sstar) & (idxx < NB), sm, 0.0))
    r2 = (r - cgt2).astype(jnp.float32)
    sum_gt = jnp.sum(jnp.where(idxx >= NB, sm, 0.0))
    tau_bits = jnp.full((1, 1), 0, jnp.int32) + (
        jnp.left_shift(bstar, SH1) | jnp.left_shift(sstar, SH2)
    )
    tau = lax.bitcast_convert_type(tau_bits, jnp.float32)
    out_ref[...] = (sum_gt + sgt2 + r2 * tau) * jnp.float32(1.0 / K)


def _finalize(cnt2, sum2, meta):
    return pl.pallas_call(
        _finalize_body,
        out_shape=jax.ShapeDtypeStruct((1, 1), jnp.float32),
    )(cnt2, sum2, meta)


_sc_hist_full = _make_sc_hist(M)
_select1_1 = _make_select1(1)
_sc_refine_1 = _make_sc_refine(1, M)


@jax.jit
def kernel(prediction, target):
    cnt, sm = _sc_losshist(prediction, target)
    out = _finalize1(cnt, sm)
    return out[0, 0]


# speedup vs baseline: 1.2205x; 1.2205x over previous
import functools

import jax
import jax.numpy as jnp
from jax import lax
from jax.experimental import pallas as pl
from jax.experimental.pallas import tpu as pltpu
from jax.experimental.pallas import tpu_sc as plsc

M, N = 128, 32768
NTOT = M * N
K = (NTOT * 10) // 100

NB = 32768
SH = 17
NW = 32
CH = 16384

_MESH = plsc.VectorSubcoreMesh(core_axis_name="c", subcore_axis_name="s")
_SC_PARAMS = pltpu.CompilerParams(needs_layout_passes=False)


def _loss_body(x_ref, t_ref, o_ref):
    x = x_ref[...]
    t = t_ref[...]
    o_ref[...] = (1.0 - t) * x + (
        jnp.log1p(jnp.exp(-jnp.abs(x))) + jnp.maximum(-x, 0.0)
    )


def _compute_loss(pred, tgt):
    blk = 4096
    return pl.pallas_call(
        _loss_body,
        grid=(N // blk,),
        in_specs=[
            pl.BlockSpec((M, blk), lambda i: (0, i)),
            pl.BlockSpec((M, blk), lambda i: (0, i)),
        ],
        out_specs=pl.BlockSpec((M, blk), lambda i: (0, i)),
        out_shape=jax.ShapeDtypeStruct((M, N), jnp.float32),
    )(pred, tgt)


def _stream(chunks, buf0, buf1, sem0, sem1, process):
    for i, ch in enumerate(chunks):
        b, s = (buf0, sem0) if i % 2 == 0 else (buf1, sem1)
        if i + 1 < len(chunks):
            nb, ns = (buf1, sem1) if i % 2 == 0 else (buf0, sem0)
            pltpu.async_copy(chunks[i + 1], nb, ns)
        pltpu.make_async_copy(chunks[0], b, s).wait()
        process(b)


@functools.partial(
    pl.kernel,
    out_type=(
        jax.ShapeDtypeStruct((NW, NB), jnp.int32),
        jax.ShapeDtypeStruct((NW, NB), jnp.float32),
    ),
    mesh=_MESH,
    compiler_params=_SC_PARAMS,
    scratch_types=[
        pltpu.VMEM((CH,), jnp.float32),
        pltpu.VMEM((CH,), jnp.float32),
        pltpu.VMEM((NB,), jnp.int32),
        pltpu.VMEM((NB,), jnp.float32),
        pltpu.SemaphoreType.DMA,
        pltpu.SemaphoreType.DMA,
    ],
)
def _sc_histsum(loss_hbm, cnt_out, sum_out, buf0, buf1, cnt, sm, sem0, sem1):
    wid = lax.axis_index("s") * 2 + lax.axis_index("c")
    row0 = wid * (M // NW)
    chunks = [loss_hbm.at[row0 + r, pl.ds(h * CH, CH)]
              for r in range(M // NW) for h in range(N // CH)]
    pltpu.async_copy(chunks[0], buf0, sem0)
    zi = jnp.zeros((16,), jnp.int32)
    zf = jnp.zeros((16,), jnp.float32)

    @plsc.parallel_loop(0, NB // 16, unroll=8)
    def _(i):
        cnt[pl.ds(i * 16, 16)] = zi
        sm[pl.ds(i * 16, 16)] = zf

    ones = jnp.ones((16,), jnp.int32)
    sh = jnp.full((16,), SH, jnp.int32)

    def process(buf):
        @plsc.parallel_loop(0, CH // 16, unroll=16)
        def _(j):
            v = buf[pl.ds(j * 16, 16)]
            u = plsc.bitcast(v, jnp.int32)
            key = jnp.right_shift(u, sh)
            plsc.addupdate_scatter(cnt, [key], ones)
            plsc.addupdate_scatter(sm, [key], v)

    _stream(chunks, buf0, buf1, sem0, sem1, process)
    pltpu.async_copy(cnt, cnt_out.at[wid], sem0)
    pltpu.async_copy(sm, sum_out.at[wid], sem1)
    pltpu.make_async_copy(cnt, cnt_out.at[wid], sem0).wait()
    pltpu.make_async_copy(sm, sum_out.at[wid], sem1).wait()


def _finalize_body(cnt_ref, sum_ref, out_ref):
    cnt = jnp.sum(cnt_ref[...], axis=0)
    sm = jnp.sum(sum_ref[...], axis=0)
    idx = lax.broadcasted_iota(jnp.int32, (NB,), 0)

    def srch(_, lohi):
        lo, hi = lohi
        mid = (lo + hi) // 2
        s = jnp.sum(jnp.where(idx >= mid, cnt, 0))
        big = s >= K
        return jnp.where(big, mid, lo), jnp.where(big, hi, mid)

    bstar, _ = lax.fori_loop(0, 15, srch, (jnp.int32(0), jnp.int32(NB)))
    cgt = jnp.sum(jnp.where(idx > bstar, cnt, 0))
    sum_gt = jnp.sum(jnp.where(idx > bstar, sm, 0.0))
    nb = jnp.sum(jnp.where(idx == bstar, cnt, 0)).astype(jnp.float32)
    r = (K - cgt).astype(jnp.float32)
    lo_arr = lax.bitcast_convert_type(
        jnp.full((1, 1), 0, jnp.int32) + jnp.left_shift(bstar, SH),
        jnp.float32)
    hi_arr = lax.bitcast_convert_type(
        jnp.full((1, 1), 0, jnp.int32) + jnp.left_shift(bstar + 1, SH),
        jnp.float32)
    est = hi_arr - (r / (2.0 * nb)) * (hi_arr - lo_arr)
    out_ref[...] = (sum_gt + r * est) * jnp.float32(1.0 / K)


def _finalize(cnt, sm):
    return pl.pallas_call(
        _finalize_body,
        out_shape=jax.ShapeDtypeStruct((1, 1), jnp.float32),
    )(cnt, sm)


@jax.jit
def kernel(prediction, target):
    loss = _compute_loss(prediction, target)
    cnt, sm = _sc_histsum(loss)
    out = _finalize(cnt, sm)
    return out[0, 0]
